# Initial kernel scaffold; baseline (speedup 1.0000x reference)
import jax
import jax.numpy as jnp
from jax.experimental import pallas as pl


def kernel(x, Wg, We, be):
    raise NotImplementedError("write your pallas kernel here")


# speedup vs baseline: 3.0691x; 3.0691x over previous
import functools

import jax
import jax.numpy as jnp
from jax.experimental import pallas as pl
from jax.experimental.pallas import tpu as pltpu

_NEG_INF = float("-inf")


def _moe_dense_body(x_ref, wg_ref, we_ref, be_ref, o_ref):
    xb = x_ref[...]
    E = wg_ref.shape[1]
    logits = jnp.dot(xb, wg_ref[...], preferred_element_type=jnp.float32)
    m = jnp.max(logits, axis=-1, keepdims=True)
    p = jnp.exp(logits - m)
    p = p / jnp.sum(p, axis=-1, keepdims=True)

    ii = jax.lax.broadcasted_iota(jnp.int32, p.shape, 1)
    m1 = jnp.max(p, axis=-1, keepdims=True)
    i1 = jnp.min(jnp.where(p == m1, ii, E), axis=-1, keepdims=True)
    p_excl = jnp.where(ii == i1, _NEG_INF, p)
    m2 = jnp.max(p_excl, axis=-1, keepdims=True)
    i2 = jnp.min(jnp.where(p_excl == m2, ii, E), axis=-1, keepdims=True)

    denom = m1 + m2 + 1e-9
    g1 = m1 / denom
    g2 = m2 / denom
    gates = jnp.where(ii == i1, g1, 0.0) + jnp.where(ii == i2, g2, 0.0)

    acc = jnp.dot(gates, be_ref[...], preferred_element_type=jnp.float32)
    for e in range(E):
        ge = gates[:, e:e + 1]
        acc = acc + ge * jnp.dot(xb, we_ref[e], preferred_element_type=jnp.float32)
    o_ref[...] = acc


@functools.partial(jax.jit, static_argnames=())
def kernel(x, Wg, We, be):
    T, D = x.shape
    E = Wg.shape[1]
    M = 512
    grid = (T // M,)
    return pl.pallas_call(
        _moe_dense_body,
        grid=grid,
        in_specs=[
            pl.BlockSpec((M, D), lambda i: (i, 0)),
            pl.BlockSpec((D, E), lambda i: (0, 0)),
            pl.BlockSpec((E, D, D), lambda i: (0, 0, 0)),
            pl.BlockSpec((E, D), lambda i: (0, 0)),
        ],
        out_specs=pl.BlockSpec((M, D), lambda i: (i, 0)),
        out_shape=jax.ShapeDtypeStruct((T, D), jnp.float32),
    )(x, Wg, We, be)
